# baseline (speedup 1.0000x reference)
import functools
import math

import jax
import jax.numpy as jnp
from jax.experimental import pallas as pl
from jax.experimental.pallas import tpu as pltpu

_C = 10
_B = 2048
_EPS = 0.05
_ROWS = 256
_K0 = 1.0 + math.exp(2.0 * _EPS)
_LN2 = math.log(2.0)


def _softmax(x):
    m = jnp.max(x, axis=1, keepdims=True)
    e = jnp.exp(x - m)
    return e / jnp.sum(e, axis=1, keepdims=True)


def _auc_kernel(ys_ref, ysa_ref, labc_ref, labr_ref, emp_ref, disc_ref,
                ee_ref, ts_ref, red_ref, w_ref):
    i = pl.program_id(0)
    nsteps = pl.num_programs(0)
    lab_row = labr_ref[...]

    @pl.when(i == 0)
    def _build_tables():
        p = _softmax(ys_ref[...])
        pa = _softmax(ysa_ref[...])
        e4p = jnp.exp(4.0 * p)
        t2 = jnp.exp(2.0 * (p - pa))
        k0col = jnp.full((_B, 1), _K0, jnp.float32)
        ee_ref[...] = jnp.concatenate([e4p, 1.0 / e4p, k0col], axis=1)
        ts_ref[...] = jnp.concatenate([t2, 1.0 / t2, k0col], axis=1)
        lab_all = labc_ref[...]
        cls = jax.lax.broadcasted_iota(jnp.int32, (1, _C), 1)
        oh_full = (lab_all == cls).astype(jnp.float32)
        ones = jnp.ones((_B, 1), jnp.float32)
        red_ref[...] = jnp.concatenate([oh_full, ones], axis=1)
        w = jnp.zeros((_B, 1), jnp.float32)
        for c in range(_C):
            n_c = jnp.sum((lab_row == c).astype(jnp.float32))
            fac_c = _LN2 / (n_c * (_B - n_c))
            w = w + jnp.where(lab_all == c, fac_c, 0.0)
        w_ref[...] = w

    @pl.when(i == 0)
    def _init_out():
        emp_ref[...] = jnp.zeros((1, 1), jnp.float32)
        disc_ref[...] = jnp.zeros((1, 1), jnp.float32)

    rows = pl.ds(i * _ROWS, _ROWS)
    lab_blk = labc_ref[rows, :]

    cls = jax.lax.broadcasted_iota(jnp.int32, (1, _C), 1)
    onehot = (lab_blk == cls).astype(jnp.float32)

    e4g = jnp.sum(onehot * ee_ref[rows, 0:_C], axis=1, keepdims=True)
    t2g = jnp.sum(onehot * ts_ref[rows, 0:_C], axis=1, keepdims=True)
    c_e = math.exp(_EPS + 4.0) / e4g
    c_e_inv = math.exp(_EPS - 4.0) * e4g
    c_s = math.exp(_EPS) / t2g
    c_s_inv = math.exp(_EPS) * t2g

    ones_col = jnp.ones((_ROWS, 1), jnp.float32)
    oh_e = jnp.concatenate([onehot * c_e, onehot * c_e_inv, ones_col], axis=1)
    oh_s = jnp.concatenate([onehot * c_s, onehot * c_s_inv, ones_col], axis=1)

    dot = functools.partial(
        jax.lax.dot_general,
        dimension_numbers=(((1,), (1,)), ((), ())),
        preferred_element_type=jnp.float32,
        precision=jax.lax.Precision.DEFAULT,
    )
    l_e = jnp.log2(dot(oh_e, ee_ref[...]))
    l_s = jnp.log2(dot(oh_s, ts_ref[...]))

    dot_red = functools.partial(
        jax.lax.dot_general,
        dimension_numbers=(((1,), (0,)), ((), ())),
        preferred_element_type=jnp.float32,
        precision=jax.lax.Precision.HIGHEST,
    )
    s_e = dot_red(l_e, red_ref[...])
    s_s = dot_red(l_s, red_ref[...])
    w_blk = w_ref[rows, :]
    own_e = jnp.sum(onehot * s_e[:, 0:_C], axis=1, keepdims=True)
    own_s = jnp.sum(onehot * s_s[:, 0:_C], axis=1, keepdims=True)
    emp = jnp.sum(w_blk * (s_e[:, _C:_C + 1] - own_e)).reshape(1, 1)
    disc = jnp.sum(w_blk * (s_s[:, _C:_C + 1] - own_s)).reshape(1, 1)

    emp_ref[...] += emp
    disc_ref[...] += disc


def kernel(y_s, y_s_adv, labels_s, y_t, y_t_adv, epoch):
    lab = labels_s.astype(jnp.int32)
    lab_col = lab.reshape(_B, 1)
    lab_row = lab.reshape(1, _B)

    grid = (_B // _ROWS,)
    emp, disc = pl.pallas_call(
        _auc_kernel,
        grid=grid,
        in_specs=[
            pl.BlockSpec((_B, _C), lambda i: (0, 0)),
            pl.BlockSpec((_B, _C), lambda i: (0, 0)),
            pl.BlockSpec((_B, 1), lambda i: (0, 0)),
            pl.BlockSpec((1, _B), lambda i: (0, 0)),
        ],
        out_specs=[
            pl.BlockSpec((1, 1), lambda i: (0, 0)),
            pl.BlockSpec((1, 1), lambda i: (0, 0)),
        ],
        out_shape=[
            jax.ShapeDtypeStruct((1, 1), jnp.float32),
            jax.ShapeDtypeStruct((1, 1), jnp.float32),
        ],
        scratch_shapes=[
            pltpu.VMEM((_B, 2 * _C + 1), jnp.float32),
            pltpu.VMEM((_B, 2 * _C + 1), jnp.float32),
            pltpu.VMEM((_B, _C + 1), jnp.float32),
            pltpu.VMEM((_B, 1), jnp.float32),
        ],
    )(y_s, y_s_adv, lab_col, lab_row)

    empirical = 0.25 * emp[0, 0]
    transfer = -0.5 * disc[0, 0]
    return (empirical, transfer)


# speedup vs baseline: 1.7391x; 1.7391x over previous
import functools
import math

import jax
import jax.numpy as jnp
from jax.experimental import pallas as pl
from jax.experimental.pallas import tpu as pltpu

_C = 10
_B = 2048
_EPS = 0.05
_ROWS = 256
_K0 = 1.0 + math.exp(2.0 * _EPS)
_LN2 = math.log(2.0)


def _softmax(x):
    m = jnp.max(x, axis=1, keepdims=True)
    e = jnp.exp(x - m)
    return e / jnp.sum(e, axis=1, keepdims=True)


def _auc_kernel(ys_ref, ysa_ref, labc_ref, labr_ref, emp_ref, disc_ref,
                ee_ref, ts_ref, red_ref, w_ref):
    i = pl.program_id(0)
    nsteps = pl.num_programs(0)
    lab_row = labr_ref[...]

    @pl.when(i == 0)
    def _build_tables():
        p = _softmax(ys_ref[...])
        pa = _softmax(ysa_ref[...])
        e4p = jnp.exp(4.0 * p)
        t2 = jnp.exp(2.0 * (p - pa))
        ee_ref[...] = jnp.concatenate([e4p, 1.0 / e4p], axis=1)
        ts_ref[...] = jnp.concatenate([t2, 1.0 / t2], axis=1)
        lab_all = labc_ref[...]
        cls = jax.lax.broadcasted_iota(jnp.int32, (1, _C), 1)
        oh_full = (lab_all == cls).astype(jnp.float32)
        ones = jnp.ones((_B, 1), jnp.float32)
        red_ref[...] = jnp.concatenate([oh_full, ones], axis=1)
        w = jnp.zeros((_B, 1), jnp.float32)
        for c in range(_C):
            n_c = jnp.sum((lab_row == c).astype(jnp.float32))
            fac_c = _LN2 / (n_c * (_B - n_c))
            w = w + jnp.where(lab_all == c, fac_c, 0.0)
        w_ref[...] = w

    @pl.when(i == 0)
    def _init_out():
        emp_ref[...] = jnp.zeros((1, 1), jnp.float32)
        disc_ref[...] = jnp.zeros((1, 1), jnp.float32)

    rows = pl.ds(i * _ROWS, _ROWS)
    lab_blk = labc_ref[rows, :]

    cls = jax.lax.broadcasted_iota(jnp.int32, (1, _C), 1)
    onehot = (lab_blk == cls).astype(jnp.float32)

    e4g = jnp.sum(onehot * ee_ref[rows, 0:_C], axis=1, keepdims=True)
    t2g = jnp.sum(onehot * ts_ref[rows, 0:_C], axis=1, keepdims=True)
    c_e = math.exp(_EPS + 4.0) / e4g
    c_e_inv = math.exp(_EPS - 4.0) * e4g
    c_s = math.exp(_EPS) / t2g
    c_s_inv = math.exp(_EPS) * t2g

    oh_e = jnp.concatenate([onehot * c_e, onehot * c_e_inv], axis=1)
    oh_s = jnp.concatenate([onehot * c_s, onehot * c_s_inv], axis=1)

    dot = functools.partial(
        jax.lax.dot_general,
        dimension_numbers=(((1,), (1,)), ((), ())),
        preferred_element_type=jnp.float32,
        precision=jax.lax.Precision.DEFAULT,
    )
    l_e = jnp.log2(_K0 + dot(oh_e, ee_ref[...]))
    l_s = jnp.log2(_K0 + dot(oh_s, ts_ref[...]))

    dot_red = functools.partial(
        jax.lax.dot_general,
        dimension_numbers=(((1,), (0,)), ((), ())),
        preferred_element_type=jnp.float32,
        precision=jax.lax.Precision.DEFAULT,
    )
    s_e = dot_red(l_e, red_ref[...])
    s_s = dot_red(l_s, red_ref[...])
    w_blk = w_ref[rows, :]
    own_e = jnp.sum(onehot * s_e[:, 0:_C], axis=1, keepdims=True)
    own_s = jnp.sum(onehot * s_s[:, 0:_C], axis=1, keepdims=True)
    emp = jnp.sum(w_blk * (s_e[:, _C:_C + 1] - own_e)).reshape(1, 1)
    disc = jnp.sum(w_blk * (s_s[:, _C:_C + 1] - own_s)).reshape(1, 1)

    emp_ref[...] += emp
    disc_ref[...] += disc


def kernel(y_s, y_s_adv, labels_s, y_t, y_t_adv, epoch):
    lab = labels_s.astype(jnp.int32)
    lab_col = lab.reshape(_B, 1)
    lab_row = lab.reshape(1, _B)

    grid = (_B // _ROWS,)
    emp, disc = pl.pallas_call(
        _auc_kernel,
        grid=grid,
        in_specs=[
            pl.BlockSpec((_B, _C), lambda i: (0, 0)),
            pl.BlockSpec((_B, _C), lambda i: (0, 0)),
            pl.BlockSpec((_B, 1), lambda i: (0, 0)),
            pl.BlockSpec((1, _B), lambda i: (0, 0)),
        ],
        out_specs=[
            pl.BlockSpec((1, 1), lambda i: (0, 0)),
            pl.BlockSpec((1, 1), lambda i: (0, 0)),
        ],
        out_shape=[
            jax.ShapeDtypeStruct((1, 1), jnp.float32),
            jax.ShapeDtypeStruct((1, 1), jnp.float32),
        ],
        scratch_shapes=[
            pltpu.VMEM((_B, 2 * _C), jnp.float32),
            pltpu.VMEM((_B, 2 * _C), jnp.float32),
            pltpu.VMEM((_B, _C + 1), jnp.float32),
            pltpu.VMEM((_B, 1), jnp.float32),
        ],
    )(y_s, y_s_adv, lab_col, lab_row)

    empirical = 0.25 * emp[0, 0]
    transfer = -0.5 * disc[0, 0]
    return (empirical, transfer)


# speedup vs baseline: 1.7987x; 1.0342x over previous
import functools
import math

import jax
import jax.numpy as jnp
from jax.experimental import pallas as pl
from jax.experimental.pallas import tpu as pltpu

_C = 10
_B = 2048
_EPS = 0.05
_ROWS = 256
_K0 = 1.0 + math.exp(2.0 * _EPS)
_LN2 = math.log(2.0)


def _softmax(x):
    m = jnp.max(x, axis=1, keepdims=True)
    e = jnp.exp(x - m)
    return e / jnp.sum(e, axis=1, keepdims=True)


def _auc_kernel(ys_ref, ysa_ref, labc_ref, labr_ref, emp_ref, disc_ref,
                ee_ref, ts_ref, red_ref, w_ref):
    i = pl.program_id(0)
    nsteps = pl.num_programs(0)
    lab_row = labr_ref[...]

    @pl.when(i == 0)
    def _build_tables():
        p = _softmax(ys_ref[...])
        pa = _softmax(ysa_ref[...])
        e4p = jnp.exp(4.0 * p)
        t2 = jnp.exp(2.0 * (p - pa))
        k0col = jnp.full((_B, 1), _K0, jnp.float32)
        ee_ref[...] = jnp.concatenate([e4p, 1.0 / e4p, k0col], axis=1)
        ts_ref[...] = jnp.concatenate([t2, 1.0 / t2, k0col], axis=1)
        lab_all = labc_ref[...]
        cls = jax.lax.broadcasted_iota(jnp.int32, (1, _C), 1)
        oh_full = (lab_all == cls).astype(jnp.float32)
        ones = jnp.ones((_B, 1), jnp.float32)
        red_ref[...] = jnp.concatenate([oh_full, ones], axis=1)
        w = jnp.zeros((_B, 1), jnp.float32)
        for c in range(_C):
            n_c = jnp.sum((lab_row == c).astype(jnp.float32))
            fac_c = _LN2 / (n_c * (_B - n_c))
            w = w + jnp.where(lab_all == c, fac_c, 0.0)
        w_ref[...] = w

    @pl.when(i == 0)
    def _init_out():
        emp_ref[...] = jnp.zeros((1, 1), jnp.float32)
        disc_ref[...] = jnp.zeros((1, 1), jnp.float32)

    rows = pl.ds(i * _ROWS, _ROWS)
    lab_blk = labc_ref[rows, :]

    cls = jax.lax.broadcasted_iota(jnp.int32, (1, _C), 1)
    onehot = (lab_blk == cls).astype(jnp.float32)

    e4g = jnp.sum(onehot * ee_ref[rows, 0:_C], axis=1, keepdims=True)
    t2g = jnp.sum(onehot * ts_ref[rows, 0:_C], axis=1, keepdims=True)
    c_e = math.exp(_EPS + 4.0) / e4g
    c_e_inv = math.exp(_EPS - 4.0) * e4g
    c_s = math.exp(_EPS) / t2g
    c_s_inv = math.exp(_EPS) * t2g

    ones_col = jnp.ones((_ROWS, 1), jnp.float32)
    oh_e = jnp.concatenate([onehot * c_e, onehot * c_e_inv, ones_col], axis=1)
    oh_s = jnp.concatenate([onehot * c_s, onehot * c_s_inv, ones_col], axis=1)

    dot = functools.partial(
        jax.lax.dot_general,
        dimension_numbers=(((1,), (1,)), ((), ())),
        preferred_element_type=jnp.float32,
        precision=jax.lax.Precision.DEFAULT,
    )
    l_e = jnp.log2(dot(oh_e, ee_ref[...]))
    l_s = jnp.log2(dot(oh_s, ts_ref[...]))

    dot_red = functools.partial(
        jax.lax.dot_general,
        dimension_numbers=(((1,), (0,)), ((), ())),
        preferred_element_type=jnp.float32,
        precision=jax.lax.Precision.DEFAULT,
    )
    s_e = dot_red(l_e, red_ref[...])
    s_s = dot_red(l_s, red_ref[...])
    w_blk = w_ref[rows, :]
    own_e = jnp.sum(onehot * s_e[:, 0:_C], axis=1, keepdims=True)
    own_s = jnp.sum(onehot * s_s[:, 0:_C], axis=1, keepdims=True)
    emp = jnp.sum(w_blk * (s_e[:, _C:_C + 1] - own_e)).reshape(1, 1)
    disc = jnp.sum(w_blk * (s_s[:, _C:_C + 1] - own_s)).reshape(1, 1)

    emp_ref[...] += emp
    disc_ref[...] += disc


def kernel(y_s, y_s_adv, labels_s, y_t, y_t_adv, epoch):
    lab = labels_s.astype(jnp.int32)
    lab_col = lab.reshape(_B, 1)
    lab_row = lab.reshape(1, _B)

    grid = (_B // _ROWS,)
    emp, disc = pl.pallas_call(
        _auc_kernel,
        grid=grid,
        in_specs=[
            pl.BlockSpec((_B, _C), lambda i: (0, 0)),
            pl.BlockSpec((_B, _C), lambda i: (0, 0)),
            pl.BlockSpec((_B, 1), lambda i: (0, 0)),
            pl.BlockSpec((1, _B), lambda i: (0, 0)),
        ],
        out_specs=[
            pl.BlockSpec((1, 1), lambda i: (0, 0)),
            pl.BlockSpec((1, 1), lambda i: (0, 0)),
        ],
        out_shape=[
            jax.ShapeDtypeStruct((1, 1), jnp.float32),
            jax.ShapeDtypeStruct((1, 1), jnp.float32),
        ],
        scratch_shapes=[
            pltpu.VMEM((_B, 2 * _C + 1), jnp.float32),
            pltpu.VMEM((_B, 2 * _C + 1), jnp.float32),
            pltpu.VMEM((_B, _C + 1), jnp.float32),
            pltpu.VMEM((_B, 1), jnp.float32),
        ],
    )(y_s, y_s_adv, lab_col, lab_row)

    empirical = 0.25 * emp[0, 0]
    transfer = -0.5 * disc[0, 0]
    return (empirical, transfer)
